# baseline (speedup 1.0000x reference)
import functools

import jax
import jax.numpy as jnp
import numpy as np
from jax import lax
from jax.experimental import pallas as pl
from jax.experimental.pallas import tpu as pltpu

_IMG = 8
_VMEM = 50 * 1024 * 1024


def _enc_kernel(x_ref, w1e_ref, w1o_ref, b1_ref, w2e_ref, w2o_ref, b2_ref, o_ref):
    H = x_ref.shape[0]
    IMG = x_ref.shape[1]
    M1 = H * IMG
    lhs = x_ref[...].reshape(M1, x_ref.shape[2])

    b1 = b1_ref[...]
    acc_e = jnp.dot(lhs, w1e_ref[...], preferred_element_type=jnp.float32)
    acc_o = jnp.dot(lhs, w1o_ref[...], preferred_element_type=jnp.float32)
    y = jnp.maximum(jnp.maximum(acc_e + b1, 0.0), jnp.maximum(acc_o + b1, 0.0))
    H2 = H // 2
    Nc1 = y.shape[1]
    pooled = jnp.max(y.reshape(H2, 2, IMG, Nc1), axis=1).astype(jnp.bfloat16)

    half = Nc1 // 2
    z16 = jnp.zeros((H2, IMG, 16), jnp.bfloat16)
    ev2 = jnp.concatenate([z16, pooled[:, :, half:]], axis=2)
    od2 = jnp.concatenate([pooled[:, :, :half], z16], axis=2)
    evod2 = jnp.concatenate([ev2, od2], axis=2)
    zrow = jnp.zeros((1, IMG, evod2.shape[2]), jnp.bfloat16)
    ep = jnp.concatenate([zrow, evod2, zrow], axis=0)
    M2 = H2 * IMG
    lhs2 = jnp.concatenate(
        [ep[d:d + H2].reshape(M2, evod2.shape[2]) for d in range(3)], axis=1)

    b2 = b2_ref[...]
    a2e = jnp.dot(lhs2, w2e_ref[...], preferred_element_type=jnp.float32)
    a2o = jnp.dot(lhs2, w2o_ref[...], preferred_element_type=jnp.float32)
    y2 = jnp.maximum(jnp.maximum(a2e + b2, 0.0), jnp.maximum(a2o + b2, 0.0))
    H4 = H2 // 2
    Nc2 = y2.shape[1]
    p2 = jnp.max(y2.reshape(H4, 2, IMG, Nc2), axis=1)
    o_ref[...] = jnp.swapaxes(p2, 0, 1).astype(jnp.bfloat16)


def _head_kernel(f_ref, o_ref, *, ns, nq, npro, way, inv_temp):
    f = f_ref[...]
    n_pad = f.shape[0]
    ff = f.astype(jnp.float32)
    sq = jnp.sum(ff * ff, axis=1, keepdims=True)

    off = (ns + nq) - (n_pad - 16)
    p16 = f[n_pad - 16:, :]
    dn = (((1,), (1,)), ((), ()))
    cross = lax.dot_general(f, p16, dn, preferred_element_type=jnp.float32)
    sqp = jnp.transpose(sq[n_pad - 16:, :])
    d2 = jnp.maximum(sq + sqp - 2.0 * cross, 0.0)
    dist = jnp.sqrt(d2)

    m0 = dist[:, off:off + way]
    m1 = dist[:, off + way:off + 2 * way]
    mins = jnp.minimum(m0, m1)

    sg = mins[:ns, :]
    qg = mins[ns:ns + nq, :]
    ab = lax.dot_general(qg, sg, dn, preferred_element_type=jnp.float32)
    sqa = jnp.sum(qg * qg, axis=1, keepdims=True)
    sqb = jnp.transpose(jnp.sum(sg * sg, axis=1, keepdims=True))
    o_ref[...] = -(sqa + sqb - 2.0 * ab) * inv_temp


def _phase_perm(w2):
    groups = list(range(0, w2, 2)) + list(range(1, w2, 2))
    return np.asarray([g * 16 + j for g in groups for j in range(16)])


def kernel(data_shot, data_query, protos,
           conv1_w_ee, conv1_w_oe, conv1_w_eo, conv1_w_oo, conv1_bias,
           conv2_w_ee, conv2_w_oe, conv2_w_eo, conv2_w_oo, conv2_bias):
    ns, C, H, W = data_shot.shape
    nq = data_query.shape[0]
    npro = protos.shape[0]
    way, ppc, temperature = 5, 2, 16.0
    N = ns + nq + npro
    n_pad = _IMG * pl.cdiv(N, _IMG)

    x = jnp.concatenate([data_shot, data_query, protos], axis=0).astype(jnp.bfloat16)
    x = jnp.transpose(x, (2, 0, 3, 1))
    xp = jnp.pad(x, ((1, 1), (0, 0), (1, 1), (0, 0)))
    K1 = (W // 2 + 1) * C
    ev = xp[:, :, 0::2, :].reshape(H + 2, N, K1)
    od = xp[:, :, 1::2, :].reshape(H + 2, N, K1)
    evod = jnp.concatenate([ev, od], axis=2)
    x6 = jnp.concatenate([evod[d:d + H] for d in range(3)], axis=2)
    if n_pad != N:
        x6 = jnp.pad(x6, ((0, 0), (0, n_pad - N), (0, 0)))

    perm = _phase_perm(W // 2)
    w1e = jnp.concatenate([conv1_w_ee[0], conv1_w_oe[0],
                           conv1_w_ee[1], conv1_w_oe[1],
                           conv1_w_ee[2], conv1_w_oe[2]], axis=0)[:, perm]
    w1o = jnp.concatenate([conv1_w_eo[0], conv1_w_oo[0],
                           conv1_w_eo[1], conv1_w_oo[1],
                           conv1_w_eo[2], conv1_w_oo[2]], axis=0)[:, perm]
    b1 = conv1_bias[:, perm]
    w2e = jnp.concatenate([conv2_w_ee[0], conv2_w_oe[0],
                           conv2_w_ee[1], conv2_w_oe[1],
                           conv2_w_ee[2], conv2_w_oe[2]], axis=0)
    w2o = jnp.concatenate([conv2_w_eo[0], conv2_w_oo[0],
                           conv2_w_eo[1], conv2_w_oo[1],
                           conv2_w_eo[2], conv2_w_oo[2]], axis=0)
    b2 = conv2_bias

    H4 = H // 4
    Nc2 = conv2_bias.shape[1]
    K6 = x6.shape[2]
    feats = pl.pallas_call(
        _enc_kernel,
        out_shape=jax.ShapeDtypeStruct((n_pad, H4, Nc2), jnp.bfloat16),
        grid=(n_pad // _IMG,),
        in_specs=[
            pl.BlockSpec((H, _IMG, K6), lambda n: (0, n, 0)),
            pl.BlockSpec(w1e.shape, lambda n: (0, 0)),
            pl.BlockSpec(w1o.shape, lambda n: (0, 0)),
            pl.BlockSpec(b1.shape, lambda n: (0, 0)),
            pl.BlockSpec(w2e.shape, lambda n: (0, 0)),
            pl.BlockSpec(w2o.shape, lambda n: (0, 0)),
            pl.BlockSpec(b2.shape, lambda n: (0, 0)),
        ],
        out_specs=pl.BlockSpec((_IMG, H4, Nc2), lambda n: (n, 0, 0)),
        compiler_params=pltpu.CompilerParams(
            dimension_semantics=("parallel",),
            vmem_limit_bytes=_VMEM),
    )(x6, w1e, w1o, b1, w2e, w2o, b2)

    D = H4 * Nc2
    feats = feats.reshape(n_pad, D)

    head = functools.partial(_head_kernel, ns=ns, nq=nq, npro=npro, way=way,
                             inv_temp=float(1.0 / temperature))
    logits = pl.pallas_call(
        head,
        out_shape=jax.ShapeDtypeStruct((nq, ns), jnp.float32),
        grid=(1,),
        in_specs=[pl.BlockSpec((n_pad, D), lambda i: (0, 0))],
        out_specs=pl.BlockSpec((nq, ns), lambda i: (0, 0)),
        compiler_params=pltpu.CompilerParams(
            dimension_semantics=("arbitrary",),
            vmem_limit_bytes=_VMEM),
    )(feats)
    return logits


# speedup vs baseline: 2.6192x; 2.6192x over previous
import functools

import jax
import jax.numpy as jnp
import numpy as np
from jax import lax
from jax.experimental import pallas as pl
from jax.experimental.pallas import tpu as pltpu

_IMG = 8
_VMEM = 50 * 1024 * 1024


def _enc_kernel(x_ref, t1_ref, b1_ref, t2_ref, b2_ref, o_ref):
    IMG = x_ref.shape[0]
    C = x_ref.shape[1]
    H = x_ref.shape[2] - 2
    W = x_ref.shape[3]
    M1 = IMG * H

    lhs = jnp.concatenate(
        [x_ref[:, c, d:d + H, :].reshape(M1, W) for d in range(3) for c in range(C)],
        axis=1)
    acc = jnp.dot(lhs, t1_ref[...], preferred_element_type=jnp.float32)
    y = jnp.maximum(acc + b1_ref[...], 0.0)
    n1 = y.shape[1] // 2
    yw = jnp.maximum(y[:, :n1], y[:, n1:])
    H2 = H // 2
    pooled = jnp.max(yw.reshape(IMG, H2, 2, n1), axis=2).astype(jnp.bfloat16)

    zrow = jnp.zeros((IMG, 1, n1), jnp.bfloat16)
    hp = jnp.concatenate([zrow, pooled, zrow], axis=1)
    M2 = IMG * H2
    lhs2 = jnp.concatenate(
        [hp[:, d:d + H2, :].reshape(M2, n1) for d in range(3)], axis=1)
    acc2 = jnp.dot(lhs2, t2_ref[...], preferred_element_type=jnp.float32)
    y2 = jnp.maximum(acc2 + b2_ref[...], 0.0)
    n2 = y2.shape[1] // 2
    y2w = jnp.maximum(y2[:, :n2], y2[:, n2:])
    H4 = H2 // 2
    o_ref[...] = jnp.max(
        y2w.reshape(IMG, H4, 2, n2), axis=2).astype(jnp.bfloat16)


def _head_kernel(f_ref, o_ref, *, ns, nq, way, inv_temp):
    f = f_ref[...]
    n_pad = f.shape[0]
    ff = f.astype(jnp.float32)
    sq = jnp.sum(ff * ff, axis=1, keepdims=True)

    off = (ns + nq) - (n_pad - 16)
    p16 = f[n_pad - 16:, :]
    dn = (((1,), (1,)), ((), ()))
    cross = lax.dot_general(f, p16, dn, preferred_element_type=jnp.float32)
    sqp = jnp.transpose(sq[n_pad - 16:, :])
    dist = jnp.sqrt(jnp.maximum(sq + sqp - 2.0 * cross, 0.0))

    mins = jnp.minimum(dist[:, off:off + way],
                       dist[:, off + way:off + 2 * way])
    sg = mins[:ns, :]
    qg = mins[ns:ns + nq, :]
    ab = lax.dot_general(qg, sg, dn, preferred_element_type=jnp.float32)
    sqa = jnp.sum(qg * qg, axis=1, keepdims=True)
    sqb = jnp.transpose(jnp.sum(sg * sg, axis=1, keepdims=True))
    o_ref[...] = -(sqa + sqb - 2.0 * ab) * inv_temp


def _col_maps(w_out, cout):
    half = w_out // 2
    wo, co = [], []
    for h in (0, 1):
        for c in range(cout):
            for q in range(half):
                wo.append(2 * q + h)
                co.append(c)
    return np.asarray(wo), np.asarray(co)


def _toeplitz(taps, w_in, w_out, cout):
    wo_map, co_map = _col_maps(w_out, cout)
    p = np.arange(w_in)
    d = np.arange(3)
    mask = (p[None, :, None] ==
            wo_map[None, None, :] + d[:, None, None] - 1).astype(np.float32)
    tapsel = taps[:, :, :, co_map].astype(jnp.float32)
    t = jnp.einsum("dpk,ydck->ycpk", jnp.asarray(mask), tapsel)
    t = t.reshape(3 * taps.shape[2] * w_in, w_out * cout)
    return t.astype(jnp.bfloat16), co_map


def kernel(data_shot, data_query, protos,
           conv1_w_ee, conv1_w_oe, conv1_w_eo, conv1_w_oo, conv1_bias,
           conv2_w_ee, conv2_w_oe, conv2_w_eo, conv2_w_oo, conv2_bias):
    ns, C, H, W = data_shot.shape
    nq = data_query.shape[0]
    npro = protos.shape[0]
    way, temperature = 5, 16.0
    hid = conv2_w_ee.shape[2] // (W // 4)
    N = ns + nq + npro
    n_pad = _IMG * pl.cdiv(N, _IMG)

    x = jnp.concatenate([data_shot, data_query, protos], axis=0).astype(jnp.bfloat16)
    xp = jnp.pad(x, ((0, n_pad - N), (0, 0), (1, 1), (0, 0)))

    taps1 = jnp.stack([conv1_w_ee[:, 0:C, 0:hid],
                       conv1_w_oe[:, 0:C, 0:hid],
                       conv1_w_ee[:, C:2 * C, 0:hid]], axis=1)
    taps2 = jnp.stack([conv2_w_ee[:, 0:hid, 0:hid],
                       conv2_w_oe[:, 0:hid, 0:hid],
                       conv2_w_ee[:, hid:2 * hid, 0:hid]], axis=1)
    t1, co1 = _toeplitz(taps1, W, W, hid)
    t2, co2 = _toeplitz(taps2, W // 2, W // 2, hid)
    b1 = conv1_bias[0, 0:hid][co1][None, :]
    b2 = conv2_bias[0, 0:hid][co2][None, :]

    H4, W4 = H // 4, W // 4
    Nc2 = W4 * hid
    feats = pl.pallas_call(
        _enc_kernel,
        out_shape=jax.ShapeDtypeStruct((n_pad, H4, Nc2), jnp.bfloat16),
        grid=(n_pad // _IMG,),
        in_specs=[
            pl.BlockSpec((_IMG, C, H + 2, W), lambda n: (n, 0, 0, 0)),
            pl.BlockSpec(t1.shape, lambda n: (0, 0)),
            pl.BlockSpec(b1.shape, lambda n: (0, 0)),
            pl.BlockSpec(t2.shape, lambda n: (0, 0)),
            pl.BlockSpec(b2.shape, lambda n: (0, 0)),
        ],
        out_specs=pl.BlockSpec((_IMG, H4, Nc2), lambda n: (n, 0, 0)),
        compiler_params=pltpu.CompilerParams(
            dimension_semantics=("parallel",),
            vmem_limit_bytes=_VMEM),
    )(xp, t1, b1, t2, b2)

    D = H4 * Nc2
    feats = feats.reshape(n_pad, D)

    head = functools.partial(_head_kernel, ns=ns, nq=nq, way=way,
                             inv_temp=float(1.0 / temperature))
    logits = pl.pallas_call(
        head,
        out_shape=jax.ShapeDtypeStruct((nq, ns), jnp.float32),
        grid=(1,),
        in_specs=[pl.BlockSpec((n_pad, D), lambda i: (0, 0))],
        out_specs=pl.BlockSpec((nq, ns), lambda i: (0, 0)),
        compiler_params=pltpu.CompilerParams(
            dimension_semantics=("arbitrary",),
            vmem_limit_bytes=_VMEM),
    )(feats)
    return logits


# speedup vs baseline: 3.5869x; 1.3695x over previous
import functools

import jax
import jax.numpy as jnp
import numpy as np
from jax import lax
from jax.experimental import pallas as pl
from jax.experimental.pallas import tpu as pltpu

_IMG = 8
_VMEM = 50 * 1024 * 1024


def _enc_kernel(x_ref, t1_ref, b1_ref, t2_ref, b2_ref, o_ref):
    IMG = x_ref.shape[0]
    C = x_ref.shape[1]
    H = x_ref.shape[2] - 2
    W = x_ref.shape[3]
    H4 = H // 4
    Mq = IMG * H4

    t1 = t1_ref[...]
    b1 = b1_ref[...]
    n1 = b1.shape[1] // 2

    def conv1_phase(j):
        lhs = jnp.concatenate(
            [x_ref[:, c, pl.ds(d + j, H4, 4), :].astype(jnp.bfloat16)
             .reshape(Mq, W)
             for d in range(3) for c in range(C)], axis=1)
        y = jnp.maximum(
            jnp.dot(lhs, t1, preferred_element_type=jnp.float32) + b1, 0.0)
        return jnp.maximum(y[:, :n1], y[:, n1:])

    pe = jnp.maximum(conv1_phase(0), conv1_phase(1)).astype(jnp.bfloat16)
    po = jnp.maximum(conv1_phase(2), conv1_phase(3)).astype(jnp.bfloat16)
    pe3 = pe.reshape(IMG, H4, n1)
    po3 = po.reshape(IMG, H4, n1)

    zrow = jnp.zeros((IMG, 1, n1), jnp.bfloat16)
    po_dn = jnp.concatenate([zrow, po3[:, :H4 - 1, :]], axis=1).reshape(Mq, n1)
    pe_up = jnp.concatenate([pe3[:, 1:, :], zrow], axis=1).reshape(Mq, n1)

    t2 = t2_ref[...]
    b2 = b2_ref[...]
    n2 = b2.shape[1] // 2

    def conv2_phase(pieces):
        lhs = jnp.concatenate(pieces, axis=1)
        y = jnp.maximum(
            jnp.dot(lhs, t2, preferred_element_type=jnp.float32) + b2, 0.0)
        return jnp.maximum(y[:, :n2], y[:, n2:])

    out = jnp.maximum(conv2_phase([po_dn, pe, po]),
                      conv2_phase([pe, po, pe_up]))
    o_ref[...] = out.reshape(IMG, H4, n2).astype(jnp.bfloat16)


def _head_kernel(f_ref, o_ref, *, ns, nq, way, inv_temp):
    f = f_ref[...]
    n_pad = f.shape[0]
    ff = f.astype(jnp.float32)
    sq = jnp.sum(ff * ff, axis=1, keepdims=True)

    off = (ns + nq) - (n_pad - 16)
    p16 = f[n_pad - 16:, :]
    dn = (((1,), (1,)), ((), ()))
    cross = lax.dot_general(f, p16, dn, preferred_element_type=jnp.float32)
    sqp = jnp.transpose(sq[n_pad - 16:, :])
    dist = jnp.sqrt(jnp.maximum(sq + sqp - 2.0 * cross, 0.0))

    mins = jnp.minimum(dist[:, off:off + way],
                       dist[:, off + way:off + 2 * way])
    sg = mins[:ns, :]
    qg = mins[ns:ns + nq, :]
    ab = lax.dot_general(qg, sg, dn, preferred_element_type=jnp.float32)
    sqa = jnp.sum(qg * qg, axis=1, keepdims=True)
    sqb = jnp.transpose(jnp.sum(sg * sg, axis=1, keepdims=True))
    o_ref[...] = -(sqa + sqb - 2.0 * ab) * inv_temp


def _col_maps(w_out, cout):
    half = w_out // 2
    wo, co = [], []
    for h in (0, 1):
        for c in range(cout):
            for q in range(half):
                wo.append(2 * q + h)
                co.append(c)
    return np.asarray(wo), np.asarray(co)


def _toeplitz(taps, w_in, w_out, cout):
    wo_map, co_map = _col_maps(w_out, cout)
    p = np.arange(w_in)
    d = np.arange(3)
    mask = (p[None, :, None] ==
            wo_map[None, None, :] + d[:, None, None] - 1).astype(np.float32)
    tapsel = taps[:, :, :, co_map].astype(jnp.float32)
    t = jnp.einsum("dpk,ydck->ycpk", jnp.asarray(mask), tapsel)
    t = t.reshape(3 * taps.shape[2] * w_in, w_out * cout)
    return t.astype(jnp.bfloat16), co_map


def kernel(data_shot, data_query, protos,
           conv1_w_ee, conv1_w_oe, conv1_w_eo, conv1_w_oo, conv1_bias,
           conv2_w_ee, conv2_w_oe, conv2_w_eo, conv2_w_oo, conv2_bias):
    ns, C, H, W = data_shot.shape
    nq = data_query.shape[0]
    npro = protos.shape[0]
    way, temperature = 5, 16.0
    hid = conv2_w_ee.shape[2] // (W // 4)
    N = ns + nq + npro
    n_pad = _IMG * pl.cdiv(N, _IMG)

    x = jnp.concatenate([data_shot, data_query, protos], axis=0)
    xp = jnp.pad(x, ((0, n_pad - N), (0, 0), (1, 1), (0, 0)))

    taps1 = jnp.stack([conv1_w_ee[:, 0:C, 0:hid],
                       conv1_w_oe[:, 0:C, 0:hid],
                       conv1_w_ee[:, C:2 * C, 0:hid]], axis=1)
    taps2 = jnp.stack([conv2_w_ee[:, 0:hid, 0:hid],
                       conv2_w_oe[:, 0:hid, 0:hid],
                       conv2_w_ee[:, hid:2 * hid, 0:hid]], axis=1)
    t1, co1 = _toeplitz(taps1, W, W, hid)
    t2, co2 = _toeplitz(taps2, W // 2, W // 2, hid)
    b1 = conv1_bias[0, 0:hid][co1][None, :]
    b2 = conv2_bias[0, 0:hid][co2][None, :]

    H4, W4 = H // 4, W // 4
    Nc2 = W4 * hid
    feats = pl.pallas_call(
        _enc_kernel,
        out_shape=jax.ShapeDtypeStruct((n_pad, H4, Nc2), jnp.bfloat16),
        grid=(n_pad // _IMG,),
        in_specs=[
            pl.BlockSpec((_IMG, C, H + 2, W), lambda n: (n, 0, 0, 0)),
            pl.BlockSpec(t1.shape, lambda n: (0, 0)),
            pl.BlockSpec(b1.shape, lambda n: (0, 0)),
            pl.BlockSpec(t2.shape, lambda n: (0, 0)),
            pl.BlockSpec(b2.shape, lambda n: (0, 0)),
        ],
        out_specs=pl.BlockSpec((_IMG, H4, Nc2), lambda n: (n, 0, 0)),
        compiler_params=pltpu.CompilerParams(
            dimension_semantics=("parallel",),
            vmem_limit_bytes=_VMEM),
    )(xp, t1, b1, t2, b2)

    D = H4 * Nc2
    feats = feats.reshape(n_pad, D)

    head = functools.partial(_head_kernel, ns=ns, nq=nq, way=way,
                             inv_temp=float(1.0 / temperature))
    logits = pl.pallas_call(
        head,
        out_shape=jax.ShapeDtypeStruct((nq, ns), jnp.float32),
        grid=(1,),
        in_specs=[pl.BlockSpec((n_pad, D), lambda i: (0, 0))],
        out_specs=pl.BlockSpec((nq, ns), lambda i: (0, 0)),
        compiler_params=pltpu.CompilerParams(
            dimension_semantics=("arbitrary",),
            vmem_limit_bytes=_VMEM),
    )(feats)
    return logits


# speedup vs baseline: 3.6138x; 1.0075x over previous
import functools

import jax
import jax.numpy as jnp
import numpy as np
from jax import lax
from jax.experimental import pallas as pl
from jax.experimental.pallas import tpu as pltpu

_IMG = 8
_VMEM = 50 * 1024 * 1024


def _enc_kernel(x_ref, t1_ref, b1_ref, t2_ref, b2_ref, o_ref):
    IMG = x_ref.shape[0]
    C = x_ref.shape[1]
    H = x_ref.shape[2]
    W = x_ref.shape[3]
    H4 = H // 4
    Mq = IMG * H4

    zrow = jnp.zeros((IMG, 1, W), jnp.bfloat16)

    def piece(c, j, d):
        start = d + j - 1
        if start < 0:
            body = x_ref[:, c, pl.ds(3, H4 - 1, 4), :].astype(jnp.bfloat16)
            return jnp.concatenate([zrow, body], axis=1)
        if start + 4 * (H4 - 1) >= H:
            body = x_ref[:, c, pl.ds(start, H4 - 1, 4), :].astype(jnp.bfloat16)
            return jnp.concatenate([body, zrow], axis=1)
        return x_ref[:, c, pl.ds(start, H4, 4), :].astype(jnp.bfloat16)

    t1 = t1_ref[...]
    b1 = b1_ref[...]
    n1 = b1.shape[1] // 2

    def conv1_phase(j):
        lhs = jnp.concatenate(
            [piece(c, j, d) for d in range(3) for c in range(C)],
            axis=2).reshape(Mq, 3 * C * W)
        y = jnp.maximum(
            jnp.dot(lhs, t1, preferred_element_type=jnp.float32) + b1, 0.0)
        return jnp.maximum(y[:, :n1], y[:, n1:])

    pe = jnp.maximum(conv1_phase(0), conv1_phase(1)).astype(jnp.bfloat16)
    po = jnp.maximum(conv1_phase(2), conv1_phase(3)).astype(jnp.bfloat16)
    pe3 = pe.reshape(IMG, H4, n1)
    po3 = po.reshape(IMG, H4, n1)

    zrow1 = jnp.zeros((IMG, 1, n1), jnp.bfloat16)
    po_dn = jnp.concatenate([zrow1, po3[:, :H4 - 1, :]], axis=1)
    pe_up = jnp.concatenate([pe3[:, 1:, :], zrow1], axis=1)

    t2 = t2_ref[...]
    b2 = b2_ref[...]
    n2 = b2.shape[1] // 2

    def conv2_phase(pieces):
        lhs = jnp.concatenate(pieces, axis=2).reshape(Mq, 3 * n1)
        y = jnp.maximum(
            jnp.dot(lhs, t2, preferred_element_type=jnp.float32) + b2, 0.0)
        return jnp.maximum(y[:, :n2], y[:, n2:])

    out = jnp.maximum(conv2_phase([po_dn, pe3, po3]),
                      conv2_phase([pe3, po3, pe_up]))
    o_ref[...] = out.reshape(IMG, H4, n2).astype(jnp.bfloat16)


def _head_kernel(f1_ref, f2_ref, f3_ref, o_ref, *, ns, nq, way, inv_temp):
    x = jnp.concatenate([f1_ref[...], f2_ref[...]], axis=0)
    p16 = f3_ref[...]
    ns_p = f1_ref.shape[0]
    xf = x.astype(jnp.float32)
    sq = jnp.sum(xf * xf, axis=1, keepdims=True)
    pf = p16.astype(jnp.float32)
    sqp = jnp.transpose(jnp.sum(pf * pf, axis=1, keepdims=True))

    dn = (((1,), (1,)), ((), ()))
    cross = lax.dot_general(x, p16, dn, preferred_element_type=jnp.float32)
    dist = jnp.sqrt(jnp.maximum(sq + sqp - 2.0 * cross, 0.0))

    mins = jnp.minimum(dist[:, 0:way], dist[:, way:2 * way])
    sg = mins[:ns, :]
    qg = mins[ns_p:ns_p + nq, :]
    ab = lax.dot_general(qg, sg, dn, preferred_element_type=jnp.float32)
    sqa = jnp.sum(qg * qg, axis=1, keepdims=True)
    sqb = jnp.transpose(jnp.sum(sg * sg, axis=1, keepdims=True))
    o_ref[...] = -(sqa + sqb - 2.0 * ab) * inv_temp


def _col_maps(w_out, cout):
    half = w_out // 2
    wo, co = [], []
    for h in (0, 1):
        for c in range(cout):
            for q in range(half):
                wo.append(2 * q + h)
                co.append(c)
    return np.asarray(wo), np.asarray(co)


def _toeplitz(taps, w_in, w_out, cout):
    wo_map, co_map = _col_maps(w_out, cout)
    p = np.arange(w_in)
    d = np.arange(3)
    mask = (p[None, :, None] ==
            wo_map[None, None, :] + d[:, None, None] - 1).astype(np.float32)
    tapsel = taps[:, :, :, co_map].astype(jnp.float32)
    t = jnp.einsum("dpk,ydck->ycpk", jnp.asarray(mask), tapsel)
    t = t.reshape(3 * taps.shape[2] * w_in, w_out * cout)
    return t.astype(jnp.bfloat16), co_map


def kernel(data_shot, data_query, protos,
           conv1_w_ee, conv1_w_oe, conv1_w_eo, conv1_w_oo, conv1_bias,
           conv2_w_ee, conv2_w_oe, conv2_w_eo, conv2_w_oo, conv2_bias):
    ns, C, H, W = data_shot.shape
    nq = data_query.shape[0]
    npro = protos.shape[0]
    way, temperature = 5, 16.0
    hid = conv2_w_ee.shape[2] // (W // 4)

    taps1 = jnp.stack([conv1_w_ee[:, 0:C, 0:hid],
                       conv1_w_oe[:, 0:C, 0:hid],
                       conv1_w_ee[:, C:2 * C, 0:hid]], axis=1)
    taps2 = jnp.stack([conv2_w_ee[:, 0:hid, 0:hid],
                       conv2_w_oe[:, 0:hid, 0:hid],
                       conv2_w_ee[:, hid:2 * hid, 0:hid]], axis=1)
    t1, co1 = _toeplitz(taps1, W, W, hid)
    t2, co2 = _toeplitz(taps2, W // 2, W // 2, hid)
    b1 = conv1_bias[0, 0:hid][co1][None, :]
    b2 = conv2_bias[0, 0:hid][co2][None, :]

    H4, W4 = H // 4, W // 4
    Nc2 = W4 * hid

    def encode(x):
        n = x.shape[0]
        blocks = pl.cdiv(n, _IMG)
        return pl.pallas_call(
            _enc_kernel,
            out_shape=jax.ShapeDtypeStruct((blocks * _IMG, H4, Nc2), jnp.bfloat16),
            grid=(blocks,),
            in_specs=[
                pl.BlockSpec((_IMG, C, H, W), lambda n: (n, 0, 0, 0)),
                pl.BlockSpec(t1.shape, lambda n: (0, 0)),
                pl.BlockSpec(b1.shape, lambda n: (0, 0)),
                pl.BlockSpec(t2.shape, lambda n: (0, 0)),
                pl.BlockSpec(b2.shape, lambda n: (0, 0)),
            ],
            out_specs=pl.BlockSpec((_IMG, H4, Nc2), lambda n: (n, 0, 0)),
            compiler_params=pltpu.CompilerParams(
                dimension_semantics=("parallel",),
                vmem_limit_bytes=_VMEM),
        )(x, t1, b1, t2, b2)

    D = H4 * Nc2
    f1 = encode(data_shot).reshape(-1, D)
    f2 = encode(data_query).reshape(-1, D)
    f3 = encode(protos).reshape(-1, D)

    head = functools.partial(_head_kernel, ns=ns, nq=nq, way=way,
                             inv_temp=float(1.0 / temperature))
    logits = pl.pallas_call(
        head,
        out_shape=jax.ShapeDtypeStruct((nq, ns), jnp.float32),
        grid=(1,),
        in_specs=[
            pl.BlockSpec(f1.shape, lambda i: (0, 0)),
            pl.BlockSpec(f2.shape, lambda i: (0, 0)),
            pl.BlockSpec(f3.shape, lambda i: (0, 0)),
        ],
        out_specs=pl.BlockSpec((nq, ns), lambda i: (0, 0)),
        compiler_params=pltpu.CompilerParams(
            dimension_semantics=("arbitrary",),
            vmem_limit_bytes=_VMEM),
    )(f1, f2, f3)
    return logits


# speedup vs baseline: 3.9719x; 1.0991x over previous
import functools

import jax
import jax.numpy as jnp
import numpy as np
from jax import lax
from jax.experimental import pallas as pl
from jax.experimental.pallas import tpu as pltpu

_IMG = 8
_VMEM = 50 * 1024 * 1024


def _enc_kernel(x_ref, t1_ref, b1_ref, t2_ref, b2_ref, o_ref):
    IMG = x_ref.shape[0]
    C = x_ref.shape[1]
    H = x_ref.shape[2]
    W = x_ref.shape[3]
    H4 = H // 4
    Mq = IMG * H4

    zrow = jnp.zeros((IMG, 1, W), jnp.bfloat16)

    def piece(c, j, d):
        start = d + j - 1
        if start < 0:
            body = x_ref[:, c, pl.ds(3, H4 - 1, 4), :].astype(jnp.bfloat16)
            return jnp.concatenate([zrow, body], axis=1)
        if start + 4 * (H4 - 1) >= H:
            body = x_ref[:, c, pl.ds(start, H4 - 1, 4), :].astype(jnp.bfloat16)
            return jnp.concatenate([body, zrow], axis=1)
        return x_ref[:, c, pl.ds(start, H4, 4), :].astype(jnp.bfloat16)

    t1 = t1_ref[...]
    b1 = b1_ref[...]
    n1 = b1.shape[1] // 2

    def conv1_phase(j):
        lhs = jnp.concatenate(
            [piece(c, j, d) for d in range(3) for c in range(C)],
            axis=2).reshape(Mq, 3 * C * W)
        y = jnp.maximum(
            jnp.dot(lhs, t1, preferred_element_type=jnp.float32) + b1, 0.0)
        return jnp.maximum(y[:, :n1], y[:, n1:])

    pe = jnp.maximum(conv1_phase(0), conv1_phase(1)).astype(jnp.bfloat16)
    po = jnp.maximum(conv1_phase(2), conv1_phase(3)).astype(jnp.bfloat16)
    pe3 = pe.reshape(IMG, H4, n1)
    po3 = po.reshape(IMG, H4, n1)

    zrow1 = jnp.zeros((IMG, 1, n1), jnp.bfloat16)
    po_dn = jnp.concatenate([zrow1, po3[:, :H4 - 1, :]], axis=1)
    pe_up = jnp.concatenate([pe3[:, 1:, :], zrow1], axis=1)

    t2 = t2_ref[...]
    b2 = b2_ref[...]
    n2 = b2.shape[1] // 2

    def conv2_phase(pieces):
        lhs = jnp.concatenate(pieces, axis=2).reshape(Mq, 3 * n1)
        y = jnp.maximum(
            jnp.dot(lhs, t2, preferred_element_type=jnp.float32) + b2, 0.0)
        return jnp.maximum(y[:, :n2], y[:, n2:])

    out = jnp.maximum(conv2_phase([po_dn, pe3, po3]),
                      conv2_phase([pe3, po3, pe_up]))
    o_ref[...] = out.reshape(IMG, H4, n2).astype(jnp.bfloat16)


def _head_kernel(f1_ref, f2_ref, f3_ref, o_ref, *, ns, nq, way, inv_temp):
    x = jnp.concatenate([f1_ref[...], f2_ref[...]], axis=0)
    p16 = f3_ref[...]
    ns_p = f1_ref.shape[0]
    xf = x.astype(jnp.float32)
    sq = jnp.sum(xf * xf, axis=1, keepdims=True)
    pf = p16.astype(jnp.float32)
    sqp = jnp.transpose(jnp.sum(pf * pf, axis=1, keepdims=True))

    dn = (((1,), (1,)), ((), ()))
    cross = lax.dot_general(x, p16, dn, preferred_element_type=jnp.float32)
    dist = jnp.sqrt(jnp.maximum(sq + sqp - 2.0 * cross, 0.0))

    mins = jnp.minimum(dist[:, 0:way], dist[:, way:2 * way])
    sg = mins[:ns, :]
    qg = mins[ns_p:ns_p + nq, :]
    ab = lax.dot_general(qg, sg, dn, preferred_element_type=jnp.float32)
    sqa = jnp.sum(qg * qg, axis=1, keepdims=True)
    sqb = jnp.transpose(jnp.sum(sg * sg, axis=1, keepdims=True))
    o_ref[...] = -(sqa + sqb - 2.0 * ab) * inv_temp


def _tbuild_kernel(ts1_ref, wo1_ref, ts2_ref, wo2_ref, t1_ref, t2_ref):
    def build(ts_ref, wo_ref, t_ref, cin, w_in):
        kb = t_ref.shape[1]
        wo = wo_ref[0:1, :]
        p = lax.broadcasted_iota(jnp.int32, (w_in, kb), 0)
        for y in range(3):
            for c in range(cin):
                acc = jnp.zeros((w_in, kb), jnp.float32)
                for d in range(3):
                    row = ts_ref[(y * 3 + d) * cin + c:(y * 3 + d) * cin + c + 1, :]
                    acc = jnp.where(p == wo + (d - 1), row.astype(jnp.float32), acc)
                r0 = (y * cin + c) * w_in
                t_ref[r0:r0 + w_in, :] = acc.astype(t_ref.dtype)

    build(ts1_ref, wo1_ref, t1_ref, ts1_ref.shape[0] // 9, t1_ref.shape[0] * 3 // ts1_ref.shape[0])
    build(ts2_ref, wo2_ref, t2_ref, ts2_ref.shape[0] // 9, t2_ref.shape[0] * 3 // ts2_ref.shape[0])


def _col_maps(w_out, cout):
    half = w_out // 2
    wo, co = [], []
    for h in (0, 1):
        for c in range(cout):
            for q in range(half):
                wo.append(2 * q + h)
                co.append(c)
    return np.asarray(wo), np.asarray(co)


def kernel(data_shot, data_query, protos,
           conv1_w_ee, conv1_w_oe, conv1_w_eo, conv1_w_oo, conv1_bias,
           conv2_w_ee, conv2_w_oe, conv2_w_eo, conv2_w_oo, conv2_bias):
    ns, C, H, W = data_shot.shape
    nq = data_query.shape[0]
    npro = protos.shape[0]
    way, temperature = 5, 16.0
    hid = conv2_w_ee.shape[2] // (W // 4)

    taps1 = jnp.stack([conv1_w_ee[:, 0:C, 0:hid],
                       conv1_w_oe[:, 0:C, 0:hid],
                       conv1_w_ee[:, C:2 * C, 0:hid]], axis=1)
    taps2 = jnp.stack([conv2_w_ee[:, 0:hid, 0:hid],
                       conv2_w_oe[:, 0:hid, 0:hid],
                       conv2_w_ee[:, hid:2 * hid, 0:hid]], axis=1)
    wo1_np, co1 = _col_maps(W, hid)
    wo2_np, co2 = _col_maps(W // 2, hid)
    K1, K2 = W * hid, (W // 2) * hid
    ts1 = taps1[:, :, :, co1].reshape(9 * C, K1)
    ts2 = taps2[:, :, :, co2].reshape(9 * hid, K2)
    wo1 = jnp.asarray(np.tile(wo1_np[None, :], (8, 1)), jnp.int32)
    wo2 = jnp.asarray(np.tile(wo2_np[None, :], (8, 1)), jnp.int32)
    t1, t2 = pl.pallas_call(
        _tbuild_kernel,
        out_shape=[jax.ShapeDtypeStruct((3 * C * W, K1), jnp.bfloat16),
                   jax.ShapeDtypeStruct((3 * hid * (W // 2), K2), jnp.bfloat16)],
        grid=(2,),
        in_specs=[
            pl.BlockSpec((9 * C, K1 // 2), lambda i: (0, i)),
            pl.BlockSpec((8, K1 // 2), lambda i: (0, i)),
            pl.BlockSpec((9 * hid, K2 // 2), lambda i: (0, i)),
            pl.BlockSpec((8, K2 // 2), lambda i: (0, i)),
        ],
        out_specs=[pl.BlockSpec((3 * C * W, K1 // 2), lambda i: (0, i)),
                   pl.BlockSpec((3 * hid * (W // 2), K2 // 2), lambda i: (0, i))],
        compiler_params=pltpu.CompilerParams(
            dimension_semantics=("parallel",),
            vmem_limit_bytes=_VMEM),
    )(ts1, wo1, ts2, wo2)
    b1 = conv1_bias[0, 0:hid][co1][None, :]
    b2 = conv2_bias[0, 0:hid][co2][None, :]

    H4, W4 = H // 4, W // 4
    Nc2 = W4 * hid

    def encode(x):
        n = x.shape[0]
        blocks = pl.cdiv(n, _IMG)
        return pl.pallas_call(
            _enc_kernel,
            out_shape=jax.ShapeDtypeStruct((blocks * _IMG, H4, Nc2), jnp.bfloat16),
            grid=(blocks,),
            in_specs=[
                pl.BlockSpec((_IMG, C, H, W), lambda n: (n, 0, 0, 0)),
                pl.BlockSpec(t1.shape, lambda n: (0, 0)),
                pl.BlockSpec(b1.shape, lambda n: (0, 0)),
                pl.BlockSpec(t2.shape, lambda n: (0, 0)),
                pl.BlockSpec(b2.shape, lambda n: (0, 0)),
            ],
            out_specs=pl.BlockSpec((_IMG, H4, Nc2), lambda n: (n, 0, 0)),
            compiler_params=pltpu.CompilerParams(
                dimension_semantics=("parallel",),
                vmem_limit_bytes=_VMEM),
        )(x, t1, b1, t2, b2)

    D = H4 * Nc2
    f1 = encode(data_shot).reshape(-1, D)
    f2 = encode(data_query).reshape(-1, D)
    f3 = encode(protos).reshape(-1, D)

    head = functools.partial(_head_kernel, ns=ns, nq=nq, way=way,
                             inv_temp=float(1.0 / temperature))
    logits = pl.pallas_call(
        head,
        out_shape=jax.ShapeDtypeStruct((nq, ns), jnp.float32),
        grid=(1,),
        in_specs=[
            pl.BlockSpec(f1.shape, lambda i: (0, 0)),
            pl.BlockSpec(f2.shape, lambda i: (0, 0)),
            pl.BlockSpec(f3.shape, lambda i: (0, 0)),
        ],
        out_specs=pl.BlockSpec((nq, ns), lambda i: (0, 0)),
        compiler_params=pltpu.CompilerParams(
            dimension_semantics=("arbitrary",),
            vmem_limit_bytes=_VMEM),
    )(f1, f2, f3)
    return logits


# speedup vs baseline: 5.5449x; 1.3960x over previous
import functools

import jax
import jax.numpy as jnp
import numpy as np
from jax import lax
from jax.experimental import pallas as pl
from jax.experimental.pallas import tpu as pltpu

_IMG = 8
_VMEM = 50 * 1024 * 1024


def _enc_kernel(x_ref, t1_ref, b1_ref, t2_ref, b2_ref, o_ref):
    IMG = x_ref.shape[0]
    C = x_ref.shape[1]
    H = x_ref.shape[2]
    W = x_ref.shape[3]
    H4 = H // 4
    Mq = IMG * H4

    zrow = jnp.zeros((IMG, 1, W), jnp.bfloat16)

    def piece(c, j, d):
        start = d + j - 1
        if start < 0:
            body = x_ref[:, c, pl.ds(3, H4 - 1, 4), :].astype(jnp.bfloat16)
            return jnp.concatenate([zrow, body], axis=1)
        if start + 4 * (H4 - 1) >= H:
            body = x_ref[:, c, pl.ds(start, H4 - 1, 4), :].astype(jnp.bfloat16)
            return jnp.concatenate([body, zrow], axis=1)
        return x_ref[:, c, pl.ds(start, H4, 4), :].astype(jnp.bfloat16)

    t1 = t1_ref[...]
    b1 = b1_ref[...]
    n1 = b1.shape[1] // 2

    def conv1_phase(j):
        lhs = jnp.concatenate(
            [piece(c, j, d) for d in range(3) for c in range(C)],
            axis=2).reshape(Mq, 3 * C * W)
        y = jnp.maximum(
            jnp.dot(lhs, t1, preferred_element_type=jnp.float32) + b1, 0.0)
        return jnp.maximum(y[:, :n1], y[:, n1:])

    pe = jnp.maximum(conv1_phase(0), conv1_phase(1)).astype(jnp.bfloat16)
    po = jnp.maximum(conv1_phase(2), conv1_phase(3)).astype(jnp.bfloat16)
    pe3 = pe.reshape(IMG, H4, n1)
    po3 = po.reshape(IMG, H4, n1)

    zrow1 = jnp.zeros((IMG, 1, n1), jnp.bfloat16)
    po_dn = jnp.concatenate([zrow1, po3[:, :H4 - 1, :]], axis=1)
    pe_up = jnp.concatenate([pe3[:, 1:, :], zrow1], axis=1)

    t2 = t2_ref[...]
    b2 = b2_ref[...]
    n2 = b2.shape[1] // 2

    def conv2_phase(pieces):
        lhs = jnp.concatenate(pieces, axis=2).reshape(Mq, 3 * n1)
        y = jnp.maximum(
            jnp.dot(lhs, t2, preferred_element_type=jnp.float32) + b2, 0.0)
        return jnp.maximum(y[:, :n2], y[:, n2:])

    out = jnp.maximum(conv2_phase([po_dn, pe3, po3]),
                      conv2_phase([pe3, po3, pe_up]))
    o_ref[...] = out.reshape(IMG, H4, n2).astype(jnp.bfloat16)


def _head_kernel(f1_ref, f2_ref, f3_ref, o_ref, *, ns, nq, way, inv_temp):
    x = jnp.concatenate([f1_ref[...], f2_ref[...]], axis=0)
    p16 = f3_ref[...]
    ns_p = f1_ref.shape[0]
    xf = x.astype(jnp.float32)
    sq = jnp.sum(xf * xf, axis=1, keepdims=True)
    pf = p16.astype(jnp.float32)
    sqp = jnp.transpose(jnp.sum(pf * pf, axis=1, keepdims=True))

    dn = (((1,), (1,)), ((), ()))
    cross = lax.dot_general(x, p16, dn, preferred_element_type=jnp.float32)
    dist = jnp.sqrt(jnp.maximum(sq + sqp - 2.0 * cross, 0.0))

    mins = jnp.minimum(dist[:, 0:way], dist[:, way:2 * way])
    sg = mins[:ns, :]
    qg = mins[ns_p:ns_p + nq, :]
    ab = lax.dot_general(qg, sg, dn, preferred_element_type=jnp.float32)
    sqa = jnp.sum(qg * qg, axis=1, keepdims=True)
    sqb = jnp.transpose(jnp.sum(sg * sg, axis=1, keepdims=True))
    o_ref[...] = -(sqa + sqb - 2.0 * ab) * inv_temp


def _tbuild_kernel(ts1_ref, wo1_ref, ts2_ref, wo2_ref, t1_ref, t2_ref):
    def build(ts_ref, wo_ref, t_ref, cin, w_in):
        kb = t_ref.shape[1]
        wo = wo_ref[0:1, :]
        p = lax.broadcasted_iota(jnp.int32, (w_in, kb), 0)
        for y in range(3):
            for c in range(cin):
                acc = jnp.zeros((w_in, kb), jnp.float32)
                for d in range(3):
                    row = ts_ref[(y * 3 + d) * cin + c:(y * 3 + d) * cin + c + 1, :]
                    acc = jnp.where(p == wo + (d - 1), row.astype(jnp.float32), acc)
                r0 = (y * cin + c) * w_in
                t_ref[r0:r0 + w_in, :] = acc.astype(t_ref.dtype)

    build(ts1_ref, wo1_ref, t1_ref, ts1_ref.shape[0] // 9, t1_ref.shape[0] * 3 // ts1_ref.shape[0])
    build(ts2_ref, wo2_ref, t2_ref, ts2_ref.shape[0] // 9, t2_ref.shape[0] * 3 // ts2_ref.shape[0])


def _col_maps(w_out, cout):
    half = w_out // 2
    wo, co = [], []
    for h in (0, 1):
        for c in range(cout):
            for q in range(half):
                wo.append(2 * q + h)
                co.append(c)
    return np.asarray(wo), np.asarray(co)


def kernel(data_shot, data_query, protos,
           conv1_w_ee, conv1_w_oe, conv1_w_eo, conv1_w_oo, conv1_bias,
           conv2_w_ee, conv2_w_oe, conv2_w_eo, conv2_w_oo, conv2_bias):
    ns, C, H, W = data_shot.shape
    nq = data_query.shape[0]
    npro = protos.shape[0]
    way, temperature = 5, 16.0
    hid = conv2_w_ee.shape[2] // (W // 4)

    taps1 = jnp.stack([conv1_w_ee[:, 0:C, 0:hid],
                       conv1_w_oe[:, 0:C, 0:hid],
                       conv1_w_ee[:, C:2 * C, 0:hid]], axis=1)
    taps2 = jnp.stack([conv2_w_ee[:, 0:hid, 0:hid],
                       conv2_w_oe[:, 0:hid, 0:hid],
                       conv2_w_ee[:, hid:2 * hid, 0:hid]], axis=1)
    wo1_np, co1 = _col_maps(W, hid)
    wo2_np, co2 = _col_maps(W // 2, hid)
    K1, K2 = W * hid, (W // 2) * hid

    def expand(taps, cin, w_out):
        half = jnp.broadcast_to(taps[..., None],
                                taps.shape + (w_out // 2,)).reshape(
                                    3, 3, cin, hid * (w_out // 2))
        return jnp.concatenate([half, half], axis=-1).reshape(
            9 * cin, hid * w_out)

    ts1 = expand(taps1, C, W)
    ts2 = expand(taps2, hid, W // 2)
    wo1 = jnp.asarray(np.tile(wo1_np[None, :], (8, 1)), jnp.int32)
    wo2 = jnp.asarray(np.tile(wo2_np[None, :], (8, 1)), jnp.int32)
    t1, t2 = pl.pallas_call(
        _tbuild_kernel,
        out_shape=[jax.ShapeDtypeStruct((3 * C * W, K1), jnp.bfloat16),
                   jax.ShapeDtypeStruct((3 * hid * (W // 2), K2), jnp.bfloat16)],
        grid=(2,),
        in_specs=[
            pl.BlockSpec((9 * C, K1 // 2), lambda i: (0, i)),
            pl.BlockSpec((8, K1 // 2), lambda i: (0, i)),
            pl.BlockSpec((9 * hid, K2 // 2), lambda i: (0, i)),
            pl.BlockSpec((8, K2 // 2), lambda i: (0, i)),
        ],
        out_specs=[pl.BlockSpec((3 * C * W, K1 // 2), lambda i: (0, i)),
                   pl.BlockSpec((3 * hid * (W // 2), K2 // 2), lambda i: (0, i))],
        compiler_params=pltpu.CompilerParams(
            dimension_semantics=("parallel",),
            vmem_limit_bytes=_VMEM),
    )(ts1, wo1, ts2, wo2)
    def expand_b(bias, w_out):
        half = jnp.broadcast_to(bias[0, 0:hid, None],
                                (hid, w_out // 2)).reshape(1, hid * (w_out // 2))
        return jnp.concatenate([half, half], axis=-1)

    b1 = expand_b(conv1_bias, W)
    b2 = expand_b(conv2_bias, W // 2)

    H4, W4 = H // 4, W // 4
    Nc2 = W4 * hid

    def encode(x):
        n = x.shape[0]
        blocks = pl.cdiv(n, _IMG)
        return pl.pallas_call(
            _enc_kernel,
            out_shape=jax.ShapeDtypeStruct((blocks * _IMG, H4, Nc2), jnp.bfloat16),
            grid=(blocks,),
            in_specs=[
                pl.BlockSpec((_IMG, C, H, W), lambda n: (n, 0, 0, 0)),
                pl.BlockSpec(t1.shape, lambda n: (0, 0)),
                pl.BlockSpec(b1.shape, lambda n: (0, 0)),
                pl.BlockSpec(t2.shape, lambda n: (0, 0)),
                pl.BlockSpec(b2.shape, lambda n: (0, 0)),
            ],
            out_specs=pl.BlockSpec((_IMG, H4, Nc2), lambda n: (n, 0, 0)),
            compiler_params=pltpu.CompilerParams(
                dimension_semantics=("parallel",),
                vmem_limit_bytes=_VMEM),
        )(x, t1, b1, t2, b2)

    D = H4 * Nc2
    f1 = encode(data_shot).reshape(-1, D)
    f2 = encode(data_query).reshape(-1, D)
    f3 = encode(protos).reshape(-1, D)

    head = functools.partial(_head_kernel, ns=ns, nq=nq, way=way,
                             inv_temp=float(1.0 / temperature))
    logits = pl.pallas_call(
        head,
        out_shape=jax.ShapeDtypeStruct((nq, ns), jnp.float32),
        grid=(1,),
        in_specs=[
            pl.BlockSpec(f1.shape, lambda i: (0, 0)),
            pl.BlockSpec(f2.shape, lambda i: (0, 0)),
            pl.BlockSpec(f3.shape, lambda i: (0, 0)),
        ],
        out_specs=pl.BlockSpec((nq, ns), lambda i: (0, 0)),
        compiler_params=pltpu.CompilerParams(
            dimension_semantics=("arbitrary",),
            vmem_limit_bytes=_VMEM),
    )(f1, f2, f3)
    return logits
